# baseline (speedup 1.0000x reference)
import jax
import jax.numpy as jnp
from jax import lax
from jax.experimental import pallas as pl
from jax.experimental.pallas import tpu as pltpu
from jax.experimental.pallas import tpu_sc as plsc

_NC = 2
_NS = 16
_NW = _NC * _NS
_L = 16

_D = 64
_DP = 128
_V = 1000001
_BATCH = 4096
_SEQ = 200
_SEQP = 208
_PLANES_W = _BATCH // _NW
_HALF = _PLANES_W // 2


def _body(ids_hbm, table_hbm, out_hbm,
          idx_half, g0, g1, c0, c1, gsem0, gsem1, wsem0, wsem1):
    wid = lax.axis_index("s") * _NC + lax.axis_index("c")
    pbase = wid * _PLANES_W
    grows = [g0, g1]
    crows = [c0, c1]
    gsem = [gsem0, gsem1]
    wsem = [wsem0, wsem1]

    def fire_gathers(p_local, b):
        row0 = idx_half.at[p_local, pl.ds(0, 128)]
        row1 = idx_half.at[p_local, pl.ds(128, _SEQP - 128)]
        pltpu.async_copy(table_hbm.at[row0], grows[b].at[pl.ds(0, 128)],
                         gsem[b])
        pltpu.async_copy(table_hbm.at[row1],
                         grows[b].at[pl.ds(128, _SEQP - 128)], gsem[b])

    def wait_g(b):
        pltpu.make_async_copy(table_hbm.at[pl.ds(0, _SEQP)], grows[b],
                              gsem[b]).wait()

    def compact(b):
        @pl.loop(0, _SEQ)
        def _rows(r):
            for i in range(_D // _L):
                sl = pl.ds(i * _L, _L)
                crows[b][r, sl] = grows[b][r, sl]

    def fire_wb(p_global, b):
        pltpu.async_copy(crows[b], out_hbm.at[p_global], wsem[b])

    def wait_wb(b):
        pltpu.make_async_copy(crows[b], out_hbm.at[0], wsem[b]).wait()

    @pl.loop(0, 2)
    def _half(h):
        off = pl.multiple_of(h * _HALF, 8)
        pltpu.sync_copy(ids_hbm.at[wid, pl.ds(off, _HALF)], idx_half)

        @pl.loop(0, _HALF)
        def _shift(r):
            for i in range(_SEQP // _L):
                sl = pl.ds(i * _L, _L)
                idx_half[r, sl] = idx_half[r, sl] + 1

        fire_gathers(0, 0)

        @pl.loop(0, _HALF // 2)
        def _pair(t):
            for u in range(2):
                b = u
                p_local = 2 * t + u
                if u == 0:
                    fire_gathers(p_local + 1, 1 - b)
                else:
                    @pl.when(t + 1 < _HALF // 2)
                    def _():
                        fire_gathers(p_local + 1, 1 - b)
                wait_g(b)

                @pl.when(h + t > 0)
                def _():
                    wait_wb(b)

                compact(b)
                fire_wb(pbase + h * _HALF + p_local, b)

    wait_wb(0)
    wait_wb(1)


def kernel(track_ids, embedding):
    b, s = track_ids.shape
    ids = jnp.pad(track_ids.astype(jnp.int32).reshape(_NW, _PLANES_W, _SEQ),
                  ((0, 0), (0, 0), (0, _SEQP - _SEQ)), constant_values=-1)
    table = jnp.pad(embedding, ((0, 0), (0, _DP - _D)))
    mesh = plsc.VectorSubcoreMesh(core_axis_name="c", subcore_axis_name="s")
    out = pl.kernel(
        _body,
        out_type=jax.ShapeDtypeStruct((_BATCH, _SEQ, _D), jnp.float32),
        mesh=mesh,
        compiler_params=pltpu.CompilerParams(use_tc_tiling_on_sc=True),
        scratch_types=[
            pltpu.VMEM((_HALF, _SEQP), jnp.int32),
            pltpu.VMEM((_SEQP, _DP), jnp.float32),
            pltpu.VMEM((_SEQP, _DP), jnp.float32),
            pltpu.VMEM((_SEQ, _D), jnp.float32),
            pltpu.VMEM((_SEQ, _D), jnp.float32),
            pltpu.SemaphoreType.DMA,
            pltpu.SemaphoreType.DMA,
            pltpu.SemaphoreType.DMA,
            pltpu.SemaphoreType.DMA,
        ],
    )(ids, table)
    return out


# speedup vs baseline: 2.8300x; 2.8300x over previous
import jax
import jax.numpy as jnp
from jax import lax
from jax.experimental import pallas as pl
from jax.experimental.pallas import tpu as pltpu
from jax.experimental.pallas import tpu_sc as plsc

_NC = 2
_NS = 16
_NW = _NC * _NS
_L = 16

_D = 64
_DP = 128
_V = 1000001
_B = 4096 * 200
_PER_W = _B // _NW
_WAVE = 128
_STAGES = _PER_W // (8 * _WAVE)


def _body(ids_hbm, table_hbm, out_hbm,
          idx_all, g0, g1, c0, c1, gsem0, gsem1, wsem0, wsem1):
    wid = lax.axis_index("s") * _NC + lax.axis_index("c")
    base = wid * _PER_W
    grows = [g0, g1]
    crows = [c0, c1]
    gsem = [gsem0, gsem1]
    wsem = [wsem0, wsem1]

    def fire_gather(s, j, b):
        pltpu.async_copy(table_hbm.at[idx_all.at[s, j]], grows[b], gsem[b])

    def wait_g(b):
        pltpu.make_async_copy(table_hbm.at[pl.ds(0, _WAVE)], grows[b],
                              gsem[b]).wait()

    def compact(b):
        @pl.loop(0, _WAVE)
        def _rows(r):
            for i in range(_D // _L):
                sl = pl.ds(i * _L, _L)
                crows[b][r, sl] = grows[b][r, sl]

    def fire_wb(w, b):
        off = pl.multiple_of(base + w * _WAVE, _WAVE)
        pltpu.async_copy(crows[b], out_hbm.at[pl.ds(off, _WAVE)], wsem[b])

    def wait_wb(b):
        pltpu.make_async_copy(crows[b], out_hbm.at[pl.ds(0, _WAVE)],
                              wsem[b]).wait()

    pltpu.sync_copy(ids_hbm.at[wid], idx_all)

    @pl.loop(0, _STAGES)
    def _shift(s):
        for j in range(8):
            for i in range(_WAVE // _L):
                sl = pl.ds(i * _L, _L)
                idx_all[s, j, sl] = idx_all[s, j, sl] + 1

    fire_gather(0, 0, 0)

    @pl.loop(0, _STAGES)
    def _stage(s):
        for j in range(8):
            b = j % 2
            if j < 7:
                fire_gather(s, j + 1, 1 - b)
            else:
                @pl.when(s + 1 < _STAGES)
                def _():
                    fire_gather(s + 1, 0, 1 - b)
            wait_g(b)
            if j >= 2:
                wait_wb(b)
            else:
                @pl.when(s > 0)
                def _():
                    wait_wb(b)
            compact(b)
            fire_wb(8 * s + j, b)

    wait_wb(0)
    wait_wb(1)


def kernel(track_ids, embedding):
    b, s = track_ids.shape
    ids = track_ids.astype(jnp.int32).reshape(_NW, _STAGES, 8, _WAVE)
    table = jnp.pad(embedding, ((0, 0), (0, _DP - _D)))
    mesh = plsc.VectorSubcoreMesh(core_axis_name="c", subcore_axis_name="s")
    out = pl.kernel(
        _body,
        out_type=jax.ShapeDtypeStruct((_B, _D), jnp.float32),
        mesh=mesh,
        compiler_params=pltpu.CompilerParams(use_tc_tiling_on_sc=True),
        scratch_types=[
            pltpu.VMEM((_STAGES, 8, _WAVE), jnp.int32),
            pltpu.VMEM((_WAVE, _DP), jnp.float32),
            pltpu.VMEM((_WAVE, _DP), jnp.float32),
            pltpu.VMEM((_WAVE, _D), jnp.float32),
            pltpu.VMEM((_WAVE, _D), jnp.float32),
            pltpu.SemaphoreType.DMA,
            pltpu.SemaphoreType.DMA,
            pltpu.SemaphoreType.DMA,
            pltpu.SemaphoreType.DMA,
        ],
    )(ids, table)
    return out.reshape(b, s, _D)
